# Initial kernel scaffold; baseline (speedup 1.0000x reference)
import jax
import jax.numpy as jnp
from jax.experimental import pallas as pl


def kernel(x, edge_index, edge_attr, batch, node_W, node_b, conv_W1, conv_b1, conv_g1, conv_be1, conv_W2, conv_b2, conv_t, ln_g, ln_b, lin_W, lin_b):
    raise NotImplementedError("write your pallas kernel here")


# speedup vs baseline: 1.0441x; 1.0441x over previous
import functools

import jax
import jax.numpy as jnp
from jax import lax
from jax.experimental import pallas as pl
from jax.experimental.pallas import tpu as pltpu
from jax.experimental.pallas import tpu_sc as plsc

N = 10000
E = 160000
D = 128
H = 128
L = 7
C = 112
G = 16

NC = 2
NS = 16
HH = H // 2
EPAD = 163840
EPW = EPAD // NS
KH = 128
SUPER = 1024
NSUP = EPW // SUPER
SUBS = SUPER // KH
BOUND = 6000
NLO = 6144
NHI = 4096
ZR = 128


def _edge_stats(z, src2, dst2, attr2, tvec, nacc):
    mesh = plsc.VectorSubcoreMesh(core_axis_name="c", subcore_axis_name="s")
    rpw = nacc // NS

    @functools.partial(
        pl.kernel,
        mesh=mesh,
        out_type=jax.ShapeDtypeStruct((NC, nacc, H), jnp.float32),
        scratch_types=[
            pltpu.VMEM((SUBS, KH), jnp.int32),
            pltpu.VMEM((SUBS, KH), jnp.int32),
            pltpu.VMEM((KH, H), jnp.float32),
            pltpu.VMEM((KH, HH), jnp.float32),
            pltpu.VMEM((KH, H), jnp.float32),
            pltpu.VMEM((ZR, H), jnp.float32),
            pltpu.VMEM((16,), jnp.float32),
            pltpu.VMEM_SHARED((nacc, H), jnp.float32),
            pltpu.SemaphoreType.DMA,
        ],
    )
    def k(z_hbm, src2_hbm, dst2_hbm, attr2_hbm, tvec_hbm, s_hbm,
          sidx, didx, zrow, arow, embuf, zbuf, tv, acc, sem):
        c = lax.axis_index("c")
        s = lax.axis_index("s")

        def zfill(i, _):
            zbuf[i // 8, pl.ds((i % 8) * 16, 16)] = jnp.zeros((16,), jnp.float32)
            return 0
        lax.fori_loop(0, ZR * 8, zfill, 0)
        for r in range(rpw // ZR):
            row0 = s * rpw + r * ZR
            pltpu.sync_copy(zbuf, acc.at[pl.ds(row0, ZR), :])

        pltpu.sync_copy(tvec_hbm, tv)
        plsc.subcore_barrier()
        t = tv[...]
        cb = c * HH

        def super_body(sup, _):
            row0 = s * (EPW // KH) + sup * SUBS
            pltpu.sync_copy(src2_hbm.at[pl.ds(row0, SUBS), :], sidx)
            pltpu.sync_copy(dst2_hbm.at[pl.ds(row0, SUBS), :], didx)

            def sub_body(sub, _):
                eb = s * EPW + sup * SUPER + sub * KH
                pltpu.sync_copy(attr2_hbm.at[c, pl.ds(eb, KH), :], arow)
                pltpu.async_copy(z_hbm.at[sidx.at[sub]], zrow, sem).wait()

                def edge_body(e, _):
                    for j in range(HH // 16):
                        m = jnp.maximum(zrow[e, pl.ds(cb + j * 16, 16)]
                                        + arow[e, pl.ds(j * 16, 16)], 0.0) + 1e-7
                        p = jnp.exp(t * m)
                        embuf[e, pl.ds(j * 16, 16)] = p
                        embuf[e, pl.ds(HH + j * 16, 16)] = m * p
                    return 0
                lax.fori_loop(0, KH, edge_body, 0)

                pltpu.sync_copy(embuf, acc.at[didx.at[sub]], add=True)
                return 0
            lax.fori_loop(0, SUBS, sub_body, 0)
            return 0
        lax.fori_loop(0, NSUP, super_body, 0)

        plsc.subcore_barrier()
        row0 = s * rpw
        pltpu.sync_copy(acc.at[pl.ds(row0, rpw), :],
                        s_hbm.at[c, pl.ds(row0, rpw), :])

    return k(z, src2, dst2, attr2, tvec)


_BN = 400
_NB = N // _BN


def _encoder(x, W, b):
    def body(xr, Wr, br, hr):
        hr[...] = jnp.dot(xr[...], Wr[...],
                          preferred_element_type=jnp.float32) + br[...]
    return pl.pallas_call(
        body,
        grid=(_NB,),
        in_specs=[pl.BlockSpec((_BN, D), lambda i: (i, 0)),
                  pl.BlockSpec((D, H), lambda i: (0, 0)),
                  pl.BlockSpec((1, H), lambda i: (0, 0))],
        out_specs=pl.BlockSpec((_BN, H), lambda i: (i, 0)),
        out_shape=jax.ShapeDtypeStruct((N, H), jnp.float32),
    )(x, W, b.reshape(1, H))


def _ln(u, g, b):
    mu = jnp.mean(u, axis=-1, keepdims=True)
    var = jnp.mean((u - mu) * (u - mu), axis=-1, keepdims=True)
    return (u - mu) / jnp.sqrt(var + 1e-5) * g + b


_LOB = BOUND // _BN


def _layer_tc(h, z, S_lo, S_hi, W1, b1, g1, be1, W2, b2, gn, bn):
    def body(hr, zr, slor, shir, W1r, b1r, g1r, be1r, W2r, b2r, gnr, bnr,
             hor, znr):
        i = pl.program_id(0)
        sv = jnp.where(i < _LOB, slor[...], shir[...])
        agg = jnp.concatenate(
            [sv[0, :, HH:] / (sv[0, :, :HH] + 1e-16),
             sv[1, :, HH:] / (sv[1, :, :HH] + 1e-16)], axis=-1)
        out = agg + zr[...]
        u = jnp.dot(out, W1r[...], preferred_element_type=jnp.float32) + b1r[...]
        u = jnp.maximum(_ln(u, g1r[...], be1r[...]), 0.0)
        v = jnp.dot(u, W2r[...], preferred_element_type=jnp.float32) + b2r[...]
        hn = hr[...] + v
        hor[...] = hn
        znr[...] = jnp.maximum(_ln(hn, gnr[...], bnr[...]), 0.0)

    return pl.pallas_call(
        body,
        grid=(_NB,),
        in_specs=[pl.BlockSpec((_BN, H), lambda i: (i, 0)),
                  pl.BlockSpec((_BN, H), lambda i: (i, 0)),
                  pl.BlockSpec((NC, _BN, H),
                               lambda i: (0, jnp.minimum(i, _LOB - 1), 0)),
                  pl.BlockSpec((NC, _BN, H),
                               lambda i: (0, jnp.maximum(i - _LOB, 0), 0)),
                  pl.BlockSpec((H, 2 * H), lambda i: (0, 0)),
                  pl.BlockSpec((1, 2 * H), lambda i: (0, 0)),
                  pl.BlockSpec((1, 2 * H), lambda i: (0, 0)),
                  pl.BlockSpec((1, 2 * H), lambda i: (0, 0)),
                  pl.BlockSpec((2 * H, H), lambda i: (0, 0)),
                  pl.BlockSpec((1, H), lambda i: (0, 0)),
                  pl.BlockSpec((1, H), lambda i: (0, 0)),
                  pl.BlockSpec((1, H), lambda i: (0, 0))],
        out_specs=[pl.BlockSpec((_BN, H), lambda i: (i, 0)),
                   pl.BlockSpec((_BN, H), lambda i: (i, 0))],
        out_shape=[jax.ShapeDtypeStruct((N, H), jnp.float32),
                   jax.ShapeDtypeStruct((N, H), jnp.float32)],
    )(h, z, S_lo, S_hi, W1, b1.reshape(1, -1), g1.reshape(1, -1),
      be1.reshape(1, -1), W2, b2.reshape(1, -1), gn.reshape(1, -1),
      bn.reshape(1, -1))


def _final_tc(hf, batch3, lin_W, lin_b):
    def body(hr, br, Wr, lbr, logits_r, pooled_r):
        i = pl.program_id(0)
        bv = br[0]
        onehot = (bv == lax.broadcasted_iota(jnp.int32, (G, 1), 0)
                  ).astype(jnp.float32)
        part = jnp.dot(onehot, hr[...], preferred_element_type=jnp.float32)

        @pl.when(i == 0)
        def _():
            pooled_r[...] = jnp.zeros_like(pooled_r)

        pooled_r[...] += part

        @pl.when(i == _NB - 1)
        def _():
            logits_r[...] = jnp.dot(pooled_r[...], Wr[...],
                                    preferred_element_type=jnp.float32) + lbr[...]

    return pl.pallas_call(
        body,
        grid=(_NB,),
        in_specs=[pl.BlockSpec((_BN, H), lambda i: (i, 0)),
                  pl.BlockSpec((1, 1, _BN), lambda i: (i, 0, 0)),
                  pl.BlockSpec((H, C), lambda i: (0, 0)),
                  pl.BlockSpec((1, C), lambda i: (0, 0))],
        out_specs=[pl.BlockSpec((G, C), lambda i: (0, 0)),
                   pl.BlockSpec((G, H), lambda i: (0, 0))],
        out_shape=[jax.ShapeDtypeStruct((G, C), jnp.float32),
                   jax.ShapeDtypeStruct((G, H), jnp.float32)],
    )(hf, batch3, lin_W, lin_b.reshape(1, C))


def kernel(x, edge_index, edge_attr, batch, node_W, node_b, conv_W1,
           conv_b1, conv_g1, conv_be1, conv_W2, conv_b2, conv_t, ln_g,
           ln_b, lin_W, lin_b):
    src = jnp.pad(edge_index[0].astype(jnp.int32), (0, EPAD - E))
    dst = jnp.pad(edge_index[1].astype(jnp.int32), (0, EPAD - E),
                  constant_values=N)
    src2 = src.reshape(EPAD // KH, KH)
    dst_lo = jnp.where(dst < BOUND, dst, NLO - 8).reshape(EPAD // KH, KH)
    dst_hi = jnp.where(dst >= BOUND, dst - BOUND, NHI - 8
                       ).reshape(EPAD // KH, KH)
    attr2 = jnp.pad(jnp.stack([edge_attr[:, :HH], edge_attr[:, HH:]]),
                    ((0, 0), (0, EPAD - E), (0, 0)))
    tvecs = jnp.broadcast_to(conv_t[:, None], (L, 16))
    batch3 = batch.astype(jnp.int32).reshape(_NB, 1, _BN)

    h = jnp.zeros((N, H), jnp.float32)
    z = _encoder(x, node_W, node_b)
    for i in range(L):
        S_lo = _edge_stats(z, src2, dst_lo, attr2, tvecs[i], NLO)
        S_hi = _edge_stats(z, src2, dst_hi, attr2, tvecs[i], NHI)
        j = i + 1 if i + 1 < L else 0
        h, z = _layer_tc(h, z, S_lo, S_hi, conv_W1[i], conv_b1[i], conv_g1[i],
                         conv_be1[i], conv_W2[i], conv_b2[i], ln_g[j], ln_b[j])
    logits, pooled = _final_tc(z, batch3, lin_W, lin_b)
    return (logits, pooled)


# speedup vs baseline: 1.9989x; 1.9145x over previous
import functools

import jax
import jax.numpy as jnp
from jax import lax
from jax.experimental import pallas as pl
from jax.experimental.pallas import tpu as pltpu
from jax.experimental.pallas import tpu_sc as plsc

N = 10000
E = 160000
D = 128
H = 128
L = 7
C = 112
G = 16

NC = 2
NS = 16
HH = H // 2
EPAD = 163840
EPW = EPAD // NS
KH = 128
SUPER = 1024
NSUP = EPW // SUPER
SUBS = SUPER // KH
BOUND = 6000
NLO = 6144
NHI = 4096
ZR = 128


def _edge_stats(z, src2, dst2, attr2, tvec, nacc):
    mesh = plsc.VectorSubcoreMesh(core_axis_name="c", subcore_axis_name="s")
    rpw = nacc // NS

    @functools.partial(
        pl.kernel,
        mesh=mesh,
        out_type=jax.ShapeDtypeStruct((NC, nacc, H), jnp.float32),
        scratch_types=[
            pltpu.VMEM((SUBS, KH), jnp.int32),
            pltpu.VMEM((SUBS, KH), jnp.int32),
            pltpu.VMEM((KH, H), jnp.float32),
            pltpu.VMEM((KH, HH), jnp.float32),
            pltpu.VMEM((KH, H), jnp.float32),
            pltpu.VMEM((ZR, H), jnp.float32),
            pltpu.VMEM((16,), jnp.float32),
            pltpu.VMEM_SHARED((nacc, H), jnp.float32),
            pltpu.SemaphoreType.DMA,
        ],
    )
    def k(z_hbm, src2_hbm, dst2_hbm, attr2_hbm, tvec_hbm, s_hbm,
          sidx, didx, zrow, arow, embuf, zbuf, tv, acc, sem):
        c = lax.axis_index("c")
        s = lax.axis_index("s")

        def zfill(i, _):
            zbuf[i // 8, pl.ds((i % 8) * 16, 16)] = jnp.zeros((16,), jnp.float32)
            return 0
        lax.fori_loop(0, ZR * 8, zfill, 0)
        for r in range(rpw // ZR):
            row0 = s * rpw + r * ZR
            pltpu.sync_copy(zbuf, acc.at[pl.ds(row0, ZR), :])

        pltpu.sync_copy(tvec_hbm, tv)
        plsc.subcore_barrier()
        t = tv[...]
        cb = c * HH

        def super_body(sup, _):
            row0 = s * (EPW // KH) + sup * SUBS
            pltpu.sync_copy(src2_hbm.at[pl.ds(row0, SUBS), :], sidx)
            pltpu.sync_copy(dst2_hbm.at[pl.ds(row0, SUBS), :], didx)

            def sub_body(sub, _):
                eb = s * EPW + sup * SUPER + sub * KH
                pltpu.sync_copy(attr2_hbm.at[c, pl.ds(eb, KH), :], arow)
                pltpu.async_copy(z_hbm.at[sidx.at[sub]], zrow, sem).wait()

                @plsc.parallel_loop(0, KH, 1, unroll=8)
                def edge_body(e):
                    for j in range(HH // 16):
                        m = jnp.maximum(zrow[e, pl.ds(cb + j * 16, 16)]
                                        + arow[e, pl.ds(j * 16, 16)], 0.0) + 1e-7
                        p = jnp.exp(t * m)
                        embuf[e, pl.ds(j * 16, 16)] = p
                        embuf[e, pl.ds(HH + j * 16, 16)] = m * p

                pltpu.sync_copy(embuf, acc.at[didx.at[sub]], add=True)
                return 0
            lax.fori_loop(0, SUBS, sub_body, 0)
            return 0
        lax.fori_loop(0, NSUP, super_body, 0)

        plsc.subcore_barrier()
        row0 = s * rpw
        pltpu.sync_copy(acc.at[pl.ds(row0, rpw), :],
                        s_hbm.at[c, pl.ds(row0, rpw), :])

    return k(z, src2, dst2, attr2, tvec)


_BN = 400
_NB = N // _BN


def _encoder(x, W, b):
    def body(xr, Wr, br, hr):
        hr[...] = jnp.dot(xr[...], Wr[...],
                          preferred_element_type=jnp.float32) + br[...]
    return pl.pallas_call(
        body,
        grid=(_NB,),
        in_specs=[pl.BlockSpec((_BN, D), lambda i: (i, 0)),
                  pl.BlockSpec((D, H), lambda i: (0, 0)),
                  pl.BlockSpec((1, H), lambda i: (0, 0))],
        out_specs=pl.BlockSpec((_BN, H), lambda i: (i, 0)),
        out_shape=jax.ShapeDtypeStruct((N, H), jnp.float32),
    )(x, W, b.reshape(1, H))


def _ln(u, g, b):
    mu = jnp.mean(u, axis=-1, keepdims=True)
    var = jnp.mean((u - mu) * (u - mu), axis=-1, keepdims=True)
    return (u - mu) / jnp.sqrt(var + 1e-5) * g + b


_LOB = BOUND // _BN


def _layer_tc(h, z, S_lo, S_hi, W1, b1, g1, be1, W2, b2, gn, bn):
    def body(hr, zr, slor, shir, W1r, b1r, g1r, be1r, W2r, b2r, gnr, bnr,
             hor, znr):
        i = pl.program_id(0)
        sv = jnp.where(i < _LOB, slor[...], shir[...])
        agg = jnp.concatenate(
            [sv[0, :, HH:] / (sv[0, :, :HH] + 1e-16),
             sv[1, :, HH:] / (sv[1, :, :HH] + 1e-16)], axis=-1)
        out = agg + zr[...]
        u = jnp.dot(out, W1r[...], preferred_element_type=jnp.float32) + b1r[...]
        u = jnp.maximum(_ln(u, g1r[...], be1r[...]), 0.0)
        v = jnp.dot(u, W2r[...], preferred_element_type=jnp.float32) + b2r[...]
        hn = hr[...] + v
        hor[...] = hn
        znr[...] = jnp.maximum(_ln(hn, gnr[...], bnr[...]), 0.0)

    return pl.pallas_call(
        body,
        grid=(_NB,),
        in_specs=[pl.BlockSpec((_BN, H), lambda i: (i, 0)),
                  pl.BlockSpec((_BN, H), lambda i: (i, 0)),
                  pl.BlockSpec((NC, _BN, H),
                               lambda i: (0, jnp.minimum(i, _LOB - 1), 0)),
                  pl.BlockSpec((NC, _BN, H),
                               lambda i: (0, jnp.maximum(i - _LOB, 0), 0)),
                  pl.BlockSpec((H, 2 * H), lambda i: (0, 0)),
                  pl.BlockSpec((1, 2 * H), lambda i: (0, 0)),
                  pl.BlockSpec((1, 2 * H), lambda i: (0, 0)),
                  pl.BlockSpec((1, 2 * H), lambda i: (0, 0)),
                  pl.BlockSpec((2 * H, H), lambda i: (0, 0)),
                  pl.BlockSpec((1, H), lambda i: (0, 0)),
                  pl.BlockSpec((1, H), lambda i: (0, 0)),
                  pl.BlockSpec((1, H), lambda i: (0, 0))],
        out_specs=[pl.BlockSpec((_BN, H), lambda i: (i, 0)),
                   pl.BlockSpec((_BN, H), lambda i: (i, 0))],
        out_shape=[jax.ShapeDtypeStruct((N, H), jnp.float32),
                   jax.ShapeDtypeStruct((N, H), jnp.float32)],
    )(h, z, S_lo, S_hi, W1, b1.reshape(1, -1), g1.reshape(1, -1),
      be1.reshape(1, -1), W2, b2.reshape(1, -1), gn.reshape(1, -1),
      bn.reshape(1, -1))


def _final_tc(hf, batch3, lin_W, lin_b):
    def body(hr, br, Wr, lbr, logits_r, pooled_r):
        i = pl.program_id(0)
        bv = br[0]
        onehot = (bv == lax.broadcasted_iota(jnp.int32, (G, 1), 0)
                  ).astype(jnp.float32)
        part = jnp.dot(onehot, hr[...], preferred_element_type=jnp.float32)

        @pl.when(i == 0)
        def _():
            pooled_r[...] = jnp.zeros_like(pooled_r)

        pooled_r[...] += part

        @pl.when(i == _NB - 1)
        def _():
            logits_r[...] = jnp.dot(pooled_r[...], Wr[...],
                                    preferred_element_type=jnp.float32) + lbr[...]

    return pl.pallas_call(
        body,
        grid=(_NB,),
        in_specs=[pl.BlockSpec((_BN, H), lambda i: (i, 0)),
                  pl.BlockSpec((1, 1, _BN), lambda i: (i, 0, 0)),
                  pl.BlockSpec((H, C), lambda i: (0, 0)),
                  pl.BlockSpec((1, C), lambda i: (0, 0))],
        out_specs=[pl.BlockSpec((G, C), lambda i: (0, 0)),
                   pl.BlockSpec((G, H), lambda i: (0, 0))],
        out_shape=[jax.ShapeDtypeStruct((G, C), jnp.float32),
                   jax.ShapeDtypeStruct((G, H), jnp.float32)],
    )(hf, batch3, lin_W, lin_b.reshape(1, C))


def kernel(x, edge_index, edge_attr, batch, node_W, node_b, conv_W1,
           conv_b1, conv_g1, conv_be1, conv_W2, conv_b2, conv_t, ln_g,
           ln_b, lin_W, lin_b):
    src = jnp.pad(edge_index[0].astype(jnp.int32), (0, EPAD - E))
    dst = jnp.pad(edge_index[1].astype(jnp.int32), (0, EPAD - E),
                  constant_values=N)
    src2 = src.reshape(EPAD // KH, KH)
    dst_lo = jnp.where(dst < BOUND, dst, NLO - 8).reshape(EPAD // KH, KH)
    dst_hi = jnp.where(dst >= BOUND, dst - BOUND, NHI - 8
                       ).reshape(EPAD // KH, KH)
    attr2 = jnp.pad(jnp.stack([edge_attr[:, :HH], edge_attr[:, HH:]]),
                    ((0, 0), (0, EPAD - E), (0, 0)))
    tvecs = jnp.broadcast_to(conv_t[:, None], (L, 16))
    batch3 = batch.astype(jnp.int32).reshape(_NB, 1, _BN)

    h = jnp.zeros((N, H), jnp.float32)
    z = _encoder(x, node_W, node_b)
    for i in range(L):
        S_lo = _edge_stats(z, src2, dst_lo, attr2, tvecs[i], NLO)
        S_hi = _edge_stats(z, src2, dst_hi, attr2, tvecs[i], NHI)
        j = i + 1 if i + 1 < L else 0
        h, z = _layer_tc(h, z, S_lo, S_hi, conv_W1[i], conv_b1[i], conv_g1[i],
                         conv_be1[i], conv_W2[i], conv_b2[i], ln_g[j], ln_b[j])
    logits, pooled = _final_tc(z, batch3, lin_W, lin_b)
    return (logits, pooled)


# speedup vs baseline: 2.2839x; 1.1426x over previous
import functools

import jax
import jax.numpy as jnp
from jax import lax
from jax.experimental import pallas as pl
from jax.experimental.pallas import tpu as pltpu
from jax.experimental.pallas import tpu_sc as plsc

N = 10000
E = 160000
D = 128
H = 128
L = 7
C = 112
G = 16

NC = 2
NS = 16
HH = H // 2
KH = 128
SUPER = 1024
SUBS = SUPER // KH
EPAD = 163840
EPW = EPAD // NS
NSUP = EPW // SUPER
BOUND = 5000
NLO = 5120
NHI = 5120
ZR = 8


def _edge_stats(z, srcI, dstI, attr2, tvec, nacc):
    mesh = plsc.VectorSubcoreMesh(core_axis_name="c", subcore_axis_name="s")
    rpw = nacc // NS

    @functools.partial(
        pl.kernel,
        mesh=mesh,
        out_type=jax.ShapeDtypeStruct((NC, nacc, H), jnp.float32),
        scratch_types=[
            pltpu.VMEM((SUBS, KH), jnp.int32),
            pltpu.VMEM((SUBS, KH), jnp.int32),
            pltpu.VMEM((KH, H), jnp.float32),
            pltpu.VMEM((KH, H), jnp.float32),
            pltpu.VMEM((KH, HH), jnp.float32),
            pltpu.VMEM((KH, HH), jnp.float32),
            pltpu.VMEM((KH, H), jnp.float32),
            pltpu.VMEM((ZR, H), jnp.float32),
            pltpu.VMEM((16,), jnp.float32),
            pltpu.VMEM_SHARED((nacc, H), jnp.float32),
            pltpu.SemaphoreType.DMA,
            pltpu.SemaphoreType.DMA,
        ],
    )
    def k(z_hbm, srcI_hbm, dstI_hbm, attr2_hbm, tvec_hbm, s_hbm,
          sidx, didx, zr0, zr1, ar0, ar1, em, zbuf, tv, acc,
          gsem, asem):
        c = lax.axis_index("c")
        s = lax.axis_index("s")
        zrs, ars = (zr0, zr1), (ar0, ar1)

        def zfill(i, _):
            zbuf[i // 8, pl.ds((i % 8) * 16, 16)] = jnp.zeros((16,), jnp.float32)
            return 0
        lax.fori_loop(0, ZR * 8, zfill, 0)
        for r in range(rpw // ZR):
            row0 = s * rpw + r * ZR
            pltpu.sync_copy(zbuf, acc.at[pl.ds(row0, ZR), :])

        pltpu.sync_copy(tvec_hbm, tv)
        plsc.subcore_barrier()
        t = tv[...]
        cb = c * HH

        def sup_body(i, _):
            row0 = s * (EPW // KH) + i * SUBS
            eb = s * EPW + i * SUPER
            pltpu.sync_copy(srcI_hbm.at[pl.ds(row0, SUBS), :], sidx)
            pltpu.sync_copy(dstI_hbm.at[pl.ds(row0, SUBS), :], didx)
            pltpu.async_copy(z_hbm.at[sidx.at[0]], zrs[0], gsem)
            pltpu.async_copy(attr2_hbm.at[c, pl.ds(eb, KH), :],
                             ars[0], asem)

            for sub in range(SUBS):
                par = sub % 2
                zr, ar = zrs[par], ars[par]
                pltpu.make_async_copy(z_hbm.at[pl.ds(0, KH), :], zr,
                                      gsem).wait()
                pltpu.make_async_copy(attr2_hbm.at[0, pl.ds(0, KH), :], ar,
                                      asem).wait()
                if sub < SUBS - 1:
                    pltpu.async_copy(z_hbm.at[sidx.at[sub + 1]],
                                     zrs[1 - par], gsem)
                    pltpu.async_copy(
                        attr2_hbm.at[c, pl.ds(eb + (sub + 1) * KH, KH), :],
                        ars[1 - par], asem)

                @plsc.parallel_loop(0, KH, 1, unroll=8)
                def edge_body(e):
                    for j in range(HH // 16):
                        m = jnp.maximum(zr[e, pl.ds(cb + j * 16, 16)]
                                        + ar[e, pl.ds(j * 16, 16)], 0.0) + 1e-7
                        p = jnp.exp(t * m)
                        em[e, pl.ds(j * 16, 16)] = p
                        em[e, pl.ds(HH + j * 16, 16)] = m * p

                pltpu.sync_copy(em, acc.at[didx.at[sub]], add=True)
            return 0
        lax.fori_loop(0, NSUP, sup_body, 0)

        plsc.subcore_barrier()
        row0 = s * rpw
        pltpu.sync_copy(acc.at[pl.ds(row0, rpw), :],
                        s_hbm.at[c, pl.ds(row0, rpw), :])

    return k(z, srcI, dstI, attr2, tvec)


_BN = 400
_NB = N // _BN


def _encoder(x, W, b):
    def body(xr, Wr, br, hr):
        hr[...] = jnp.dot(xr[...], Wr[...],
                          preferred_element_type=jnp.float32) + br[...]
    return pl.pallas_call(
        body,
        grid=(_NB,),
        in_specs=[pl.BlockSpec((_BN, D), lambda i: (i, 0)),
                  pl.BlockSpec((D, H), lambda i: (0, 0)),
                  pl.BlockSpec((1, H), lambda i: (0, 0))],
        out_specs=pl.BlockSpec((_BN, H), lambda i: (i, 0)),
        out_shape=jax.ShapeDtypeStruct((N, H), jnp.float32),
    )(x, W, b.reshape(1, H))


def _ln(u, g, b):
    mu = jnp.mean(u, axis=-1, keepdims=True)
    var = jnp.mean((u - mu) * (u - mu), axis=-1, keepdims=True)
    return (u - mu) / jnp.sqrt(var + 1e-5) * g + b


def _layer_tc(h, z, S, W1, b1, g1, be1, W2, b2, gn, bn):
    def body(hr, zr, sr, W1r, b1r, g1r, be1r, W2r, b2r, gnr, bnr,
             hor, znr):
        sv = sr[...]
        agg = jnp.concatenate(
            [sv[0, :, HH:] / (sv[0, :, :HH] + 1e-16),
             sv[1, :, HH:] / (sv[1, :, :HH] + 1e-16)], axis=-1)
        out = agg + zr[...]
        u = jnp.dot(out, W1r[...], preferred_element_type=jnp.float32) + b1r[...]
        u = jnp.maximum(_ln(u, g1r[...], be1r[...]), 0.0)
        v = jnp.dot(u, W2r[...], preferred_element_type=jnp.float32) + b2r[...]
        hn = hr[...] + v
        hor[...] = hn
        znr[...] = jnp.maximum(_ln(hn, gnr[...], bnr[...]), 0.0)

    return pl.pallas_call(
        body,
        grid=(_NB,),
        in_specs=[pl.BlockSpec((_BN, H), lambda i: (i, 0)),
                  pl.BlockSpec((_BN, H), lambda i: (i, 0)),
                  pl.BlockSpec((NC, _BN, H), lambda i: (0, i, 0)),
                  pl.BlockSpec((H, 2 * H), lambda i: (0, 0)),
                  pl.BlockSpec((1, 2 * H), lambda i: (0, 0)),
                  pl.BlockSpec((1, 2 * H), lambda i: (0, 0)),
                  pl.BlockSpec((1, 2 * H), lambda i: (0, 0)),
                  pl.BlockSpec((2 * H, H), lambda i: (0, 0)),
                  pl.BlockSpec((1, H), lambda i: (0, 0)),
                  pl.BlockSpec((1, H), lambda i: (0, 0)),
                  pl.BlockSpec((1, H), lambda i: (0, 0))],
        out_specs=[pl.BlockSpec((_BN, H), lambda i: (i, 0)),
                   pl.BlockSpec((_BN, H), lambda i: (i, 0))],
        out_shape=[jax.ShapeDtypeStruct((N, H), jnp.float32),
                   jax.ShapeDtypeStruct((N, H), jnp.float32)],
    )(h, z, S, W1, b1.reshape(1, -1), g1.reshape(1, -1),
      be1.reshape(1, -1), W2, b2.reshape(1, -1), gn.reshape(1, -1),
      bn.reshape(1, -1))


def _final_tc(hf, batch3, lin_W, lin_b):
    def body(hr, br, Wr, lbr, logits_r, pooled_r):
        i = pl.program_id(0)
        bv = br[0]
        onehot = (bv == lax.broadcasted_iota(jnp.int32, (G, 1), 0)
                  ).astype(jnp.float32)
        part = jnp.dot(onehot, hr[...], preferred_element_type=jnp.float32)

        @pl.when(i == 0)
        def _():
            pooled_r[...] = jnp.zeros_like(pooled_r)

        pooled_r[...] += part

        @pl.when(i == _NB - 1)
        def _():
            logits_r[...] = jnp.dot(pooled_r[...], Wr[...],
                                    preferred_element_type=jnp.float32) + lbr[...]

    return pl.pallas_call(
        body,
        grid=(_NB,),
        in_specs=[pl.BlockSpec((_BN, H), lambda i: (i, 0)),
                  pl.BlockSpec((1, 1, _BN), lambda i: (i, 0, 0)),
                  pl.BlockSpec((H, C), lambda i: (0, 0)),
                  pl.BlockSpec((1, C), lambda i: (0, 0))],
        out_specs=[pl.BlockSpec((G, C), lambda i: (0, 0)),
                   pl.BlockSpec((G, H), lambda i: (0, 0))],
        out_shape=[jax.ShapeDtypeStruct((G, C), jnp.float32),
                   jax.ShapeDtypeStruct((G, H), jnp.float32)],
    )(hf, batch3, lin_W, lin_b.reshape(1, C))


def kernel(x, edge_index, edge_attr, batch, node_W, node_b, conv_W1,
           conv_b1, conv_g1, conv_be1, conv_W2, conv_b2, conv_t, ln_g,
           ln_b, lin_W, lin_b):
    src = jnp.pad(edge_index[0].astype(jnp.int32), (0, EPAD - E))
    dst = jnp.pad(edge_index[1].astype(jnp.int32), (0, EPAD - E),
                  constant_values=N)
    srcI = src.reshape(EPAD // KH, KH)
    dst_lo = jnp.where(dst < BOUND, dst, NLO - 8).reshape(EPAD // KH, KH)
    dst_hi = jnp.where(dst >= BOUND, dst - BOUND, NHI - 8
                       ).reshape(EPAD // KH, KH)
    attr2 = jnp.pad(jnp.stack([edge_attr[:, :HH], edge_attr[:, HH:]]),
                    ((0, 0), (0, EPAD - E), (0, 0)))
    tvecs = jnp.broadcast_to(conv_t[:, None], (L, 16))
    batch3 = batch.astype(jnp.int32).reshape(_NB, 1, _BN)

    h = jnp.zeros((N, H), jnp.float32)
    z = _encoder(x, node_W, node_b)
    for i in range(L):
        S_lo = _edge_stats(z, srcI, dst_lo, attr2, tvecs[i], NLO)
        S_hi = _edge_stats(z, srcI, dst_hi, attr2, tvecs[i], NHI)
        S = jnp.concatenate([S_lo[:, :BOUND], S_hi[:, :N - BOUND]], axis=1)
        j = i + 1 if i + 1 < L else 0
        h, z = _layer_tc(h, z, S, conv_W1[i], conv_b1[i], conv_g1[i],
                         conv_be1[i], conv_W2[i], conv_b2[i], ln_g[j], ln_b[j])
    logits, pooled = _final_tc(z, batch3, lin_W, lin_b)
    return (logits, pooled)


# speedup vs baseline: 2.3083x; 1.0107x over previous
import functools

import jax
import jax.numpy as jnp
from jax import lax
from jax.experimental import pallas as pl
from jax.experimental.pallas import tpu as pltpu
from jax.experimental.pallas import tpu_sc as plsc

N = 10000
E = 160000
D = 128
H = 128
L = 7
C = 112
G = 16

NC = 2
NS = 16
HH = H // 2
KH = 128
SUPER = 1024
SUBS = SUPER // KH
EPAD = 163840
EPW = EPAD // NS
NSUP = EPW // SUPER
BOUND = 5000
NLO = 5120
NHI = 5120
ZR = 8


def _edge_stats(z, srcI, dstI, attr2, tvec, nacc):
    mesh = plsc.VectorSubcoreMesh(core_axis_name="c", subcore_axis_name="s")
    rpw = nacc // NS

    @functools.partial(
        pl.kernel,
        mesh=mesh,
        out_type=jax.ShapeDtypeStruct((NC, nacc, H), jnp.float32),
        scratch_types=[
            pltpu.VMEM((SUBS, KH), jnp.int32),
            pltpu.VMEM((SUBS, KH), jnp.int32),
            pltpu.VMEM((KH, H), jnp.float32),
            pltpu.VMEM((KH, H), jnp.float32),
            pltpu.VMEM((KH, HH), jnp.float32),
            pltpu.VMEM((KH, H), jnp.float32),
            pltpu.VMEM((KH, H), jnp.float32),
            pltpu.VMEM((ZR, H), jnp.float32),
            pltpu.VMEM((16,), jnp.float32),
            pltpu.VMEM_SHARED((nacc, H), jnp.float32),
            pltpu.SemaphoreType.DMA,
            pltpu.SemaphoreType.DMA,
            pltpu.SemaphoreType.DMA,
        ],
    )
    def k(z_hbm, srcI_hbm, dstI_hbm, attr2_hbm, tvec_hbm, s_hbm,
          sidx, didx, zr0, zr1, ar, em0, em1, zbuf, tv, acc,
          gsem, asem, ssem):
        c = lax.axis_index("c")
        s = lax.axis_index("s")
        zrs, ems = (zr0, zr1), (em0, em1)

        def zfill(i, _):
            zbuf[i // 8, pl.ds((i % 8) * 16, 16)] = jnp.zeros((16,), jnp.float32)
            return 0
        lax.fori_loop(0, ZR * 8, zfill, 0)
        for r in range(rpw // ZR):
            row0 = s * rpw + r * ZR
            pltpu.sync_copy(zbuf, acc.at[pl.ds(row0, ZR), :])

        pltpu.sync_copy(tvec_hbm, tv)
        plsc.subcore_barrier()
        t = tv[...]
        cb = c * HH

        def sup_body(i, _):
            row0 = s * (EPW // KH) + i * SUBS
            eb = s * EPW + i * SUPER
            pltpu.sync_copy(srcI_hbm.at[pl.ds(row0, SUBS), :], sidx)
            pltpu.sync_copy(dstI_hbm.at[pl.ds(row0, SUBS), :], didx)
            pltpu.async_copy(z_hbm.at[sidx.at[0]], zrs[0], gsem)

            for sub in range(SUBS):
                par = sub % 2
                zr, em = zrs[par], ems[par]
                pltpu.sync_copy(attr2_hbm.at[c, pl.ds(eb + sub * KH, KH), :],
                                ar)
                pltpu.make_async_copy(z_hbm.at[pl.ds(0, KH), :], zr,
                                      gsem).wait()
                if sub < SUBS - 1:
                    pltpu.async_copy(z_hbm.at[sidx.at[sub + 1]],
                                     zrs[1 - par], gsem)

                if sub >= 2:
                    pltpu.make_async_copy(z_hbm.at[pl.ds(0, KH), :], em,
                                          ssem).wait()
                else:
                    @pl.when(i > 0)
                    def _():
                        pltpu.make_async_copy(z_hbm.at[pl.ds(0, KH), :], em,
                                              ssem).wait()

                @plsc.parallel_loop(0, KH, 1, unroll=8)
                def edge_body(e):
                    for j in range(HH // 16):
                        m = jnp.maximum(zr[e, pl.ds(cb + j * 16, 16)]
                                        + ar[e, pl.ds(j * 16, 16)], 0.0) + 1e-7
                        p = jnp.exp(t * m)
                        em[e, pl.ds(j * 16, 16)] = p
                        em[e, pl.ds(HH + j * 16, 16)] = m * p

                pltpu.async_copy(em, acc.at[didx.at[sub]], ssem, add=True)
            return 0
        lax.fori_loop(0, NSUP, sup_body, 0)

        for em in ems:
            pltpu.make_async_copy(z_hbm.at[pl.ds(0, KH), :], em, ssem).wait()

        plsc.subcore_barrier()
        row0 = s * rpw
        pltpu.sync_copy(acc.at[pl.ds(row0, rpw), :],
                        s_hbm.at[c, pl.ds(row0, rpw), :])

    return k(z, srcI, dstI, attr2, tvec)


_BN = 400
_NB = N // _BN


def _encoder(x, W, b):
    def body(xr, Wr, br, hr):
        hr[...] = jnp.dot(xr[...], Wr[...],
                          preferred_element_type=jnp.float32) + br[...]
    return pl.pallas_call(
        body,
        grid=(_NB,),
        in_specs=[pl.BlockSpec((_BN, D), lambda i: (i, 0)),
                  pl.BlockSpec((D, H), lambda i: (0, 0)),
                  pl.BlockSpec((1, H), lambda i: (0, 0))],
        out_specs=pl.BlockSpec((_BN, H), lambda i: (i, 0)),
        out_shape=jax.ShapeDtypeStruct((N, H), jnp.float32),
    )(x, W, b.reshape(1, H))


def _ln(u, g, b):
    mu = jnp.mean(u, axis=-1, keepdims=True)
    var = jnp.mean((u - mu) * (u - mu), axis=-1, keepdims=True)
    return (u - mu) / jnp.sqrt(var + 1e-5) * g + b


def _layer_tc(h, z, S, W1, b1, g1, be1, W2, b2, gn, bn):
    def body(hr, zr, sr, W1r, b1r, g1r, be1r, W2r, b2r, gnr, bnr,
             hor, znr):
        sv = sr[...]
        agg = jnp.concatenate(
            [sv[0, :, HH:] / (sv[0, :, :HH] + 1e-16),
             sv[1, :, HH:] / (sv[1, :, :HH] + 1e-16)], axis=-1)
        out = agg + zr[...]
        u = jnp.dot(out, W1r[...], preferred_element_type=jnp.float32) + b1r[...]
        u = jnp.maximum(_ln(u, g1r[...], be1r[...]), 0.0)
        v = jnp.dot(u, W2r[...], preferred_element_type=jnp.float32) + b2r[...]
        hn = hr[...] + v
        hor[...] = hn
        znr[...] = jnp.maximum(_ln(hn, gnr[...], bnr[...]), 0.0)

    return pl.pallas_call(
        body,
        grid=(_NB,),
        in_specs=[pl.BlockSpec((_BN, H), lambda i: (i, 0)),
                  pl.BlockSpec((_BN, H), lambda i: (i, 0)),
                  pl.BlockSpec((NC, _BN, H), lambda i: (0, i, 0)),
                  pl.BlockSpec((H, 2 * H), lambda i: (0, 0)),
                  pl.BlockSpec((1, 2 * H), lambda i: (0, 0)),
                  pl.BlockSpec((1, 2 * H), lambda i: (0, 0)),
                  pl.BlockSpec((1, 2 * H), lambda i: (0, 0)),
                  pl.BlockSpec((2 * H, H), lambda i: (0, 0)),
                  pl.BlockSpec((1, H), lambda i: (0, 0)),
                  pl.BlockSpec((1, H), lambda i: (0, 0)),
                  pl.BlockSpec((1, H), lambda i: (0, 0))],
        out_specs=[pl.BlockSpec((_BN, H), lambda i: (i, 0)),
                   pl.BlockSpec((_BN, H), lambda i: (i, 0))],
        out_shape=[jax.ShapeDtypeStruct((N, H), jnp.float32),
                   jax.ShapeDtypeStruct((N, H), jnp.float32)],
    )(h, z, S, W1, b1.reshape(1, -1), g1.reshape(1, -1),
      be1.reshape(1, -1), W2, b2.reshape(1, -1), gn.reshape(1, -1),
      bn.reshape(1, -1))


def _final_tc(hf, batch3, lin_W, lin_b):
    def body(hr, br, Wr, lbr, logits_r, pooled_r):
        i = pl.program_id(0)
        bv = br[0]
        onehot = (bv == lax.broadcasted_iota(jnp.int32, (G, 1), 0)
                  ).astype(jnp.float32)
        part = jnp.dot(onehot, hr[...], preferred_element_type=jnp.float32)

        @pl.when(i == 0)
        def _():
            pooled_r[...] = jnp.zeros_like(pooled_r)

        pooled_r[...] += part

        @pl.when(i == _NB - 1)
        def _():
            logits_r[...] = jnp.dot(pooled_r[...], Wr[...],
                                    preferred_element_type=jnp.float32) + lbr[...]

    return pl.pallas_call(
        body,
        grid=(_NB,),
        in_specs=[pl.BlockSpec((_BN, H), lambda i: (i, 0)),
                  pl.BlockSpec((1, 1, _BN), lambda i: (i, 0, 0)),
                  pl.BlockSpec((H, C), lambda i: (0, 0)),
                  pl.BlockSpec((1, C), lambda i: (0, 0))],
        out_specs=[pl.BlockSpec((G, C), lambda i: (0, 0)),
                   pl.BlockSpec((G, H), lambda i: (0, 0))],
        out_shape=[jax.ShapeDtypeStruct((G, C), jnp.float32),
                   jax.ShapeDtypeStruct((G, H), jnp.float32)],
    )(hf, batch3, lin_W, lin_b.reshape(1, C))


def kernel(x, edge_index, edge_attr, batch, node_W, node_b, conv_W1,
           conv_b1, conv_g1, conv_be1, conv_W2, conv_b2, conv_t, ln_g,
           ln_b, lin_W, lin_b):
    src = jnp.pad(edge_index[0].astype(jnp.int32), (0, EPAD - E))
    dst = jnp.pad(edge_index[1].astype(jnp.int32), (0, EPAD - E),
                  constant_values=N)
    srcI = src.reshape(EPAD // KH, KH)
    dst_lo = jnp.where(dst < BOUND, dst, NLO - 8).reshape(EPAD // KH, KH)
    dst_hi = jnp.where(dst >= BOUND, dst - BOUND, NHI - 8
                       ).reshape(EPAD // KH, KH)
    attr2 = jnp.pad(jnp.stack([edge_attr[:, :HH], edge_attr[:, HH:]]),
                    ((0, 0), (0, EPAD - E), (0, 0)))
    tvecs = jnp.broadcast_to(conv_t[:, None], (L, 16))
    batch3 = batch.astype(jnp.int32).reshape(_NB, 1, _BN)

    h = jnp.zeros((N, H), jnp.float32)
    z = _encoder(x, node_W, node_b)
    for i in range(L):
        S_lo = _edge_stats(z, srcI, dst_lo, attr2, tvecs[i], NLO)
        S_hi = _edge_stats(z, srcI, dst_hi, attr2, tvecs[i], NHI)
        S = jnp.concatenate([S_lo[:, :BOUND], S_hi[:, :N - BOUND]], axis=1)
        j = i + 1 if i + 1 < L else 0
        h, z = _layer_tc(h, z, S, conv_W1[i], conv_b1[i], conv_g1[i],
                         conv_be1[i], conv_W2[i], conv_b2[i], ln_g[j], ln_b[j])
    logits, pooled = _final_tc(z, batch3, lin_W, lin_b)
    return (logits, pooled)
